# Initial kernel scaffold; baseline (speedup 1.0000x reference)
import jax
import jax.numpy as jnp
from jax.experimental import pallas as pl


def kernel(pos_u, pos_v, neg_v, u_weight, v_weight):
    raise NotImplementedError("write your pallas kernel here")


# speedup vs baseline: 1.5735x; 1.5735x over previous
import functools

import jax
import jax.numpy as jnp
from jax import lax
from jax.experimental import pallas as pl
from jax.experimental.pallas import tpu as pltpu
from jax.experimental.pallas import tpu_sc as plsc

EMB_DIM = 64
BATCH = 16384
NEG = 5

NUM_CORES = 2
NUM_SUBCORES = 16
NUM_WORKERS = NUM_CORES * NUM_SUBCORES
ROWS_PER_WORKER = BATCH // NUM_WORKERS
CHUNK = 128
NCHUNKS = ROWS_PER_WORKER // CHUNK
LANES = 16


def _sc_dots_kernel(pos_u_hbm, pos_v_hbm, negf_hbm, uw_hbm, vw_hbm,
                    pos_out, neg_out,
                    idxu, idxv, idxn, urows, vrows, nrows, pdots, ndots,
                    sem):
    wid = lax.axis_index("s") * NUM_CORES + lax.axis_index("c")
    iota = lax.iota(jnp.int32, LANES)

    def chunk_body(chunk, _):
        base = wid * ROWS_PER_WORKER + chunk * CHUNK

        pltpu.sync_copy(pos_u_hbm.at[pl.ds(base, CHUNK)], idxu)
        pltpu.sync_copy(pos_v_hbm.at[pl.ds(base, CHUNK)], idxv)
        for j in range(NEG):
            pltpu.sync_copy(
                negf_hbm.at[pl.ds(base * NEG + j * CHUNK, CHUNK)],
                idxn.at[j])

        cps = [pltpu.async_copy(uw_hbm.at[idxu], urows, sem),
               pltpu.async_copy(vw_hbm.at[idxv], vrows, sem)]
        for j in range(NEG):
            cps.append(pltpu.async_copy(
                vw_hbm.at[idxn.at[j]],
                nrows.at[pl.ds(j * CHUNK, CHUNK)], sem))
        for cp in cps:
            cp.wait()

        def group_body(g, _):
            r0 = g * LANES
            row = r0 + iota
            nrow = [row * NEG + j for j in range(NEG)]
            acc_p = jnp.zeros((LANES,), jnp.float32)
            acc_n = [jnp.zeros((LANES,), jnp.float32) for _ in range(NEG)]
            for c in range(EMB_DIM):
                col = jnp.full((LANES,), c, jnp.int32)
                uc = plsc.load_gather(urows, [row, col])
                vc = plsc.load_gather(vrows, [row, col])
                acc_p = acc_p + uc * vc
                for j in range(NEG):
                    nc = plsc.load_gather(nrows, [nrow[j], col])
                    acc_n[j] = acc_n[j] + uc * nc
            pdots[pl.ds(r0, LANES)] = acc_p
            for j in range(NEG):
                ndots[pl.ds(j * CHUNK + r0, LANES)] = acc_n[j]
            return 0

        lax.fori_loop(0, CHUNK // LANES, group_body, 0)

        pltpu.sync_copy(pdots, pos_out.at[pl.ds(base, CHUNK)])
        pltpu.sync_copy(ndots, neg_out.at[pl.ds(base * NEG, CHUNK * NEG)])
        return 0

    lax.fori_loop(0, NCHUNKS, chunk_body, 0)


_sc_dots = functools.partial(
    pl.kernel,
    mesh=plsc.VectorSubcoreMesh(core_axis_name="c", subcore_axis_name="s"),
    out_type=[jax.ShapeDtypeStruct((BATCH,), jnp.float32),
              jax.ShapeDtypeStruct((BATCH * NEG,), jnp.float32)],
    scratch_types=[
        pltpu.VMEM((CHUNK,), jnp.int32),
        pltpu.VMEM((CHUNK,), jnp.int32),
        pltpu.VMEM((NEG, CHUNK), jnp.int32),
        pltpu.VMEM((CHUNK, EMB_DIM), jnp.float32),
        pltpu.VMEM((CHUNK, EMB_DIM), jnp.float32),
        pltpu.VMEM((CHUNK * NEG, EMB_DIM), jnp.float32),
        pltpu.VMEM((CHUNK,), jnp.float32),
        pltpu.VMEM((CHUNK * NEG,), jnp.float32),
        pltpu.SemaphoreType.DMA,
    ],
    compiler_params=pltpu.CompilerParams(
        needs_layout_passes=False, use_tc_tiling_on_sc=False),
)(_sc_dots_kernel)


def _reduce_body(p_ref, n_ref, o_ref):
    s = jnp.sum(jax.nn.log_sigmoid(p_ref[...]))
    s = s + jnp.sum(jax.nn.log_sigmoid(-n_ref[...]))
    o_ref[...] = jnp.broadcast_to(-s, (1, 1))


def kernel(pos_u, pos_v, neg_v, u_weight, v_weight):
    pos_u = pos_u.astype(jnp.int32)
    pos_v = pos_v.astype(jnp.int32)
    neg_flat = neg_v.astype(jnp.int32).reshape(BATCH * NEG)

    pos_dots, neg_dots = _sc_dots(pos_u, pos_v, neg_flat, u_weight, v_weight)

    out = pl.pallas_call(
        _reduce_body,
        out_shape=jax.ShapeDtypeStruct((1, 1), jnp.float32),
    )(pos_dots.reshape(BATCH // 128, 128),
      neg_dots.reshape(BATCH * NEG // 128, 128))
    return out[0, 0]
